# Initial kernel scaffold; baseline (speedup 1.0000x reference)
import jax
import jax.numpy as jnp
from jax.experimental import pallas as pl


def kernel(z, edge_index, edge_type, rel_emb):
    raise NotImplementedError("write your pallas kernel here")


# speedup vs baseline: 1.1102x; 1.1102x over previous
import functools

import jax
import jax.numpy as jnp
from jax import lax
from jax.experimental import pallas as pl
from jax.experimental.pallas import tpu as pltpu
from jax.experimental.pallas import tpu_sc as plsc

_N_EDGES = 320000
_HIDDEN = 128
_NREL = 200
_NC = 2
_NS = 16
_NW = _NC * _NS
_EPW = _N_EDGES // _NW
_C = 80
_NCHUNK = _EPW // _C


def _sc_score(src, dst, typ, z, rel):
    mesh = plsc.VectorSubcoreMesh(core_axis_name="c", subcore_axis_name="s")

    @functools.partial(
        pl.kernel,
        mesh=mesh,
        compiler_params=pltpu.CompilerParams(needs_layout_passes=False),
        out_type=jax.ShapeDtypeStruct((_N_EDGES,), jnp.float32),
        scratch_types=[
            pltpu.VMEM((_NREL, _HIDDEN), jnp.float32),
            pltpu.VMEM((_C,), jnp.int32),
            pltpu.VMEM((_C,), jnp.int32),
            pltpu.VMEM((_C,), jnp.int32),
            pltpu.VMEM((_C, _HIDDEN), jnp.float32),
            pltpu.VMEM((_C, _HIDDEN), jnp.float32),
            pltpu.VMEM((_C,), jnp.float32),
            pltpu.SemaphoreType.DMA,
            pltpu.SemaphoreType.DMA,
        ],
    )
    def k(src_hbm, dst_hbm, typ_hbm, z_hbm, rel_hbm, out_hbm,
          rel_v, si_v, di_v, ti_v, sr_v, dr_v, ob_v, sem_s, sem_d):
        wid = lax.axis_index("s") * _NC + lax.axis_index("c")
        base = wid * _EPW
        pltpu.sync_copy(rel_hbm, rel_v)

        def chunk(kk, carry):
            off = base + kk * _C
            pltpu.sync_copy(src_hbm.at[pl.ds(off, _C)], si_v)
            pltpu.sync_copy(dst_hbm.at[pl.ds(off, _C)], di_v)
            pltpu.sync_copy(typ_hbm.at[pl.ds(off, _C)], ti_v)
            cs = pltpu.async_copy(z_hbm.at[si_v], sr_v, sem_s)
            cd = pltpu.async_copy(z_hbm.at[di_v], dr_v, sem_d)
            cs.wait()
            cd.wait()

            def group(g, c2):
                row_idx = g * 16 + lax.iota(jnp.int32, 16)
                ty_vec = ti_v[pl.ds(g * 16, 16)]

                def hstep(h, hc):
                    acc, hv = hc
                    s = plsc.load_gather(sr_v, [row_idx, hv])
                    d = plsc.load_gather(dr_v, [row_idx, hv])
                    r = plsc.load_gather(rel_v, [ty_vec, hv])
                    return acc + s * d * r, hv + 1

                acc, _ = lax.fori_loop(
                    0, _HIDDEN, hstep,
                    (jnp.zeros((16,), jnp.float32), jnp.zeros((16,), jnp.int32)),
                    unroll=8)
                ob_v[pl.ds(g * 16, 16)] = acc
                return c2

            lax.fori_loop(0, _C // 16, group, 0)
            pltpu.sync_copy(ob_v, out_hbm.at[pl.ds(off, _C)])
            return carry

        lax.fori_loop(0, _NCHUNK, chunk, 0)

    return k(src, dst, typ, z, rel)


def kernel(z, edge_index, edge_type, rel_emb):
    ei = edge_index.astype(jnp.int32)
    return _sc_score(ei[0], ei[1], edge_type.astype(jnp.int32),
                     z.astype(jnp.float32), rel_emb.astype(jnp.float32))


# speedup vs baseline: 1.2177x; 1.0968x over previous
import functools

import jax
import jax.numpy as jnp
from jax import lax
from jax.experimental import pallas as pl
from jax.experimental.pallas import tpu as pltpu
from jax.experimental.pallas import tpu_sc as plsc

_N_EDGES = 320000
_HIDDEN = 128
_NREL = 200
_NC = 2
_NS = 16
_NW = _NC * _NS
_EPW = _N_EDGES // _NW
_C = 400
_NCHUNK = _EPW // _C


def _sc_score(src, dst, typ, z, rel):
    mesh = plsc.VectorSubcoreMesh(core_axis_name="c", subcore_axis_name="s")

    @functools.partial(
        pl.kernel,
        mesh=mesh,
        compiler_params=pltpu.CompilerParams(needs_layout_passes=False),
        out_type=jax.ShapeDtypeStruct((_N_EDGES,), jnp.float32),
        scratch_types=[
            pltpu.VMEM((_NREL, _HIDDEN), jnp.float32),
            pltpu.VMEM((_C,), jnp.int32),
            pltpu.VMEM((_C,), jnp.int32),
            pltpu.VMEM((_C,), jnp.int32),
            pltpu.VMEM((_C, _HIDDEN), jnp.float32),
            pltpu.VMEM((_C, _HIDDEN), jnp.float32),
            pltpu.VMEM((_C,), jnp.float32),
            pltpu.SemaphoreType.DMA,
            pltpu.SemaphoreType.DMA,
            pltpu.SemaphoreType.DMA,
        ],
    )
    def k(src_hbm, dst_hbm, typ_hbm, z_hbm, rel_hbm, out_hbm,
          rel_v, si_v, di_v, ti_v, sr_v, dr_v, ob_v, sem_i, sem_s, sem_d):
        wid = lax.axis_index("s") * _NC + lax.axis_index("c")
        base = wid * _EPW
        pltpu.sync_copy(rel_hbm, rel_v)

        def chunk(kk, carry):
            off = base + kk * _C
            ci1 = pltpu.async_copy(src_hbm.at[pl.ds(off, _C)], si_v, sem_i)
            ci2 = pltpu.async_copy(dst_hbm.at[pl.ds(off, _C)], di_v, sem_i)
            ci3 = pltpu.async_copy(typ_hbm.at[pl.ds(off, _C)], ti_v, sem_i)
            ci1.wait()
            ci2.wait()
            ci3.wait()
            cs = pltpu.async_copy(z_hbm.at[si_v], sr_v, sem_s)
            cd = pltpu.async_copy(z_hbm.at[di_v], dr_v, sem_d)
            cs.wait()
            cd.wait()

            def group(g, c2):
                row_idx = g * 16 + lax.iota(jnp.int32, 16)
                ty_vec = ti_v[pl.ds(g * 16, 16)]

                def hstep(h, hc):
                    acc, hv = hc
                    s = plsc.load_gather(sr_v, [row_idx, hv])
                    d = plsc.load_gather(dr_v, [row_idx, hv])
                    r = plsc.load_gather(rel_v, [ty_vec, hv])
                    return acc + s * d * r, hv + 1

                acc, _ = lax.fori_loop(
                    0, _HIDDEN, hstep,
                    (jnp.zeros((16,), jnp.float32), jnp.zeros((16,), jnp.int32)),
                    unroll=8)
                ob_v[pl.ds(g * 16, 16)] = acc
                return c2

            lax.fori_loop(0, _C // 16, group, 0)
            pltpu.sync_copy(ob_v, out_hbm.at[pl.ds(off, _C)])
            return carry

        lax.fori_loop(0, _NCHUNK, chunk, 0)

    return k(src, dst, typ, z, rel)


def kernel(z, edge_index, edge_type, rel_emb):
    ei = edge_index.astype(jnp.int32)
    return _sc_score(ei[0], ei[1], edge_type.astype(jnp.int32),
                     z.astype(jnp.float32), rel_emb.astype(jnp.float32))


# speedup vs baseline: 3.4042x; 2.7956x over previous
import functools

import jax
import jax.numpy as jnp
from jax import lax
from jax.experimental import pallas as pl
from jax.experimental.pallas import tpu as pltpu
from jax.experimental.pallas import tpu_sc as plsc

_N_EDGES = 320000
_HIDDEN = 128
_NC = 2
_NS = 16
_NW = _NC * _NS
_EPW = _N_EDGES // _NW
_C = 80
_NCHUNK = _EPW // _C


def _sc_score(src, dst, typ, z, rel):
    mesh = plsc.VectorSubcoreMesh(core_axis_name="c", subcore_axis_name="s")

    @functools.partial(
        pl.kernel,
        mesh=mesh,
        compiler_params=pltpu.CompilerParams(needs_layout_passes=False),
        out_type=jax.ShapeDtypeStruct((_N_EDGES,), jnp.float32),
        scratch_types=[
            pltpu.VMEM((_C,), jnp.int32),
            pltpu.VMEM((_C,), jnp.int32),
            pltpu.VMEM((_C,), jnp.int32),
            pltpu.VMEM((_C, _HIDDEN), jnp.float32),
            pltpu.VMEM((_C, _HIDDEN), jnp.float32),
            pltpu.VMEM((_C, _HIDDEN), jnp.float32),
            pltpu.VMEM((_C,), jnp.float32),
            pltpu.SemaphoreType.DMA,
            pltpu.SemaphoreType.DMA,
            pltpu.SemaphoreType.DMA,
            pltpu.SemaphoreType.DMA,
        ],
    )
    def k(src_hbm, dst_hbm, typ_hbm, z_hbm, rel_hbm, out_hbm,
          si_v, di_v, ti_v, sr_v, dr_v, rr_v, ob_v,
          sem_i, sem_s, sem_d, sem_r):
        wid = lax.axis_index("s") * _NC + lax.axis_index("c")
        base = wid * _EPW
        lane = lax.iota(jnp.int32, 16)

        def chunk(kk, carry):
            off = base + kk * _C
            ci1 = pltpu.async_copy(src_hbm.at[pl.ds(off, _C)], si_v, sem_i)
            ci2 = pltpu.async_copy(dst_hbm.at[pl.ds(off, _C)], di_v, sem_i)
            ci3 = pltpu.async_copy(typ_hbm.at[pl.ds(off, _C)], ti_v, sem_i)
            ci1.wait()
            ci2.wait()
            ci3.wait()
            cs = pltpu.async_copy(z_hbm.at[si_v], sr_v, sem_s)
            cd = pltpu.async_copy(z_hbm.at[di_v], dr_v, sem_d)
            cr = pltpu.async_copy(rel_hbm.at[ti_v], rr_v, sem_r)
            cs.wait()
            cd.wait()
            cr.wait()

            def group(g, c2):
                res = jnp.zeros((16,), jnp.float32)
                for el in range(16):
                    e = g * 16 + el
                    acc = jnp.zeros((16,), jnp.float32)
                    for j in range(_HIDDEN // 16):
                        s = sr_v[e, pl.ds(j * 16, 16)]
                        d = dr_v[e, pl.ds(j * 16, 16)]
                        r = rr_v[e, pl.ds(j * 16, 16)]
                        acc = acc + s * d * r
                    res = jnp.where(lane == el, jnp.sum(acc), res)
                ob_v[pl.ds(g * 16, 16)] = res
                return c2

            lax.fori_loop(0, _C // 16, group, 0)
            pltpu.sync_copy(ob_v, out_hbm.at[pl.ds(off, _C)])
            return carry

        lax.fori_loop(0, _NCHUNK, chunk, 0)

    return k(src, dst, typ, z, rel)


def kernel(z, edge_index, edge_type, rel_emb):
    ei = edge_index.astype(jnp.int32)
    return _sc_score(ei[0], ei[1], edge_type.astype(jnp.int32),
                     z.astype(jnp.float32), rel_emb.astype(jnp.float32))


# speedup vs baseline: 3.7717x; 1.1080x over previous
import functools

import jax
import jax.numpy as jnp
from jax import lax
from jax.experimental import pallas as pl
from jax.experimental.pallas import tpu as pltpu
from jax.experimental.pallas import tpu_sc as plsc

_N_EDGES = 320000
_HIDDEN = 128
_NC = 2
_NS = 16
_NW = _NC * _NS
_EPW = _N_EDGES // _NW
_C = 80
_NCHUNK = _EPW // _C


def _sc_score(src, dst, typ, z, rel):
    mesh = plsc.VectorSubcoreMesh(core_axis_name="c", subcore_axis_name="s")

    @functools.partial(
        pl.kernel,
        mesh=mesh,
        compiler_params=pltpu.CompilerParams(needs_layout_passes=False),
        out_type=jax.ShapeDtypeStruct((_N_EDGES,), jnp.float32),
        scratch_types=[
            pltpu.VMEM((2, 3, _C), jnp.int32),
            pltpu.VMEM((2, _C, _HIDDEN), jnp.float32),
            pltpu.VMEM((2, _C, _HIDDEN), jnp.float32),
            pltpu.VMEM((2, _C, _HIDDEN), jnp.float32),
            pltpu.VMEM((_EPW,), jnp.float32),
            pltpu.SemaphoreType.DMA,
            pltpu.SemaphoreType.DMA,
            pltpu.SemaphoreType.DMA,
            pltpu.SemaphoreType.DMA,
        ],
    )
    def k(src_hbm, dst_hbm, typ_hbm, z_hbm, rel_hbm, out_hbm,
          iv_v, sr_v, dr_v, rr_v, ob_v, si0, si1, sg0, sg1):
        sem_i = (si0, si1)
        sem_g = (sg0, sg1)
        wid = lax.axis_index("s") * _NC + lax.axis_index("c")
        base = wid * _EPW
        lane = lax.iota(jnp.int32, 16)

        def issue_idx(kk, b):
            off = base + kk * _C
            pltpu.async_copy(src_hbm.at[pl.ds(off, _C)], iv_v.at[b, 0], sem_i[b])
            pltpu.async_copy(dst_hbm.at[pl.ds(off, _C)], iv_v.at[b, 1], sem_i[b])
            pltpu.async_copy(typ_hbm.at[pl.ds(off, _C)], iv_v.at[b, 2], sem_i[b])

        def wait_idx(b):
            for j in range(3):
                pltpu.make_async_copy(
                    src_hbm.at[pl.ds(0, _C)], iv_v.at[b, j], sem_i[b]).wait()

        def issue_gather(b):
            pltpu.async_copy(z_hbm.at[iv_v.at[b, 0]], sr_v.at[b], sem_g[b])
            pltpu.async_copy(z_hbm.at[iv_v.at[b, 1]], dr_v.at[b], sem_g[b])
            pltpu.async_copy(rel_hbm.at[iv_v.at[b, 2]], rr_v.at[b], sem_g[b])

        def wait_gather(b):
            for buf in (sr_v, dr_v, rr_v):
                pltpu.make_async_copy(
                    z_hbm.at[pl.ds(0, _C)], buf.at[b], sem_g[b]).wait()

        issue_idx(0, 0)
        issue_idx(1, 1)
        wait_idx(0)
        issue_gather(0)

        def pair(kk2, carry):
            for b in (0, 1):
                kk = kk2 * 2 + b

                @pl.when(kk < _NCHUNK)
                def _():
                    wait_gather(b)

                    @pl.when(kk + 2 < _NCHUNK)
                    def _():
                        issue_idx(kk + 2, b)

                    @pl.when(kk + 1 < _NCHUNK)
                    def _():
                        wait_idx(1 - b)
                        issue_gather(1 - b)

                    obase = kk * _C

                    def group(g, c2):
                        res = jnp.zeros((16,), jnp.float32)
                        for el in range(16):
                            e = g * 16 + el
                            acc = jnp.zeros((16,), jnp.float32)
                            for j in range(_HIDDEN // 16):
                                s = sr_v[b, e, pl.ds(j * 16, 16)]
                                d = dr_v[b, e, pl.ds(j * 16, 16)]
                                r = rr_v[b, e, pl.ds(j * 16, 16)]
                                acc = acc + s * d * r
                            res = jnp.where(lane == el, jnp.sum(acc), res)
                        ob_v[pl.ds(obase + g * 16, 16)] = res
                        return c2

                    lax.fori_loop(0, _C // 16, group, 0)

            return carry

        lax.fori_loop(0, (_NCHUNK + 1) // 2, pair, 0)
        pltpu.sync_copy(ob_v, out_hbm.at[pl.ds(base, _EPW)])

    return k(src, dst, typ, z, rel)


def kernel(z, edge_index, edge_type, rel_emb):
    ei = edge_index.astype(jnp.int32)
    return _sc_score(ei[0], ei[1], edge_type.astype(jnp.int32),
                     z.astype(jnp.float32), rel_emb.astype(jnp.float32))
